# Initial kernel scaffold; baseline (speedup 1.0000x reference)
import jax
import jax.numpy as jnp
from jax.experimental import pallas as pl


def kernel(Q, K, V, W_out):
    raise NotImplementedError("write your pallas kernel here")


# speedup vs baseline: 278.0565x; 278.0565x over previous
import jax
import jax.numpy as jnp
from jax.experimental import pallas as pl
from jax.experimental.pallas import tpu as pltpu

ETA = 0.05
LAMBDA_BASE = 0.01
ALPHA = 0.1
TOPK = 32
THETA = 2.0 ** 16
ACT_THRESH = 0.3


def _k1_rope_count(q_ref, cos_ref, sin_ref, qr_ref, cnt_ref):
    bsz, nh, tc1, n = q_ref.shape
    q = q_ref[...]
    c = cos_ref[...]
    s = sin_ref[...]
    q2 = q.reshape(bsz * nh * tc1, n)
    rm = pltpu.roll(q2, n - 1, 1).reshape(q.shape)
    rp = pltpu.roll(q2, 1, 1).reshape(q.shape)
    lane = jax.lax.broadcasted_iota(jnp.int32, q.shape, 3)
    even = (lane % 2) == 0
    qrot = jnp.where(even, -rm, rp)
    qr = q * c[None, None] + qrot * s[None, None]
    qr_ref[...] = jnp.swapaxes(qr, 0, 1)
    pos = jnp.sum((qr > 0).astype(jnp.int32), axis=(0, 1, 3))
    cnt_ref[...] = pos.reshape(1, 1, tc1)


def _k2_scan(x_ref, flags_ref, state_ref, yagg_ref, sigma_ref, h_ref):
    tb = pl.program_id(1)
    hh = pl.program_id(2)
    _, bsz, tc2, n = x_ref.shape

    @pl.when(tb == 0)
    def _():
        sigma_ref[hh] = jnp.zeros((n, n), jnp.float32)
        h_ref[hh] = jnp.zeros((n, n), jnp.float32)

    @pl.when(hh == 0)
    def _():
        yagg_ref[...] = jnp.zeros_like(yagg_ref)

    st = state_ref[tb]

    @pl.when(st == 1)
    def _():
        x = x_ref[0].reshape(bsz * tc2, n)
        y = jnp.dot(x, sigma_ref[hh], preferred_element_type=jnp.float32)
        yagg_ref[...] += y.reshape(1, bsz, tc2, n)

    @pl.when(st == 2)
    def _():
        def step(t, carry):
            x_t = jnp.concatenate(
                [x_ref[0, b, t, :].reshape(1, n) for b in range(bsz)], axis=0)
            y = jax.lax.dot_general(
                x_t, sigma_ref[hh], (((1,), (0,)), ((), ())),
                preferred_element_type=jnp.float32,
                precision=jax.lax.Precision.HIGHEST)
            for b in range(bsz):
                yagg_ref[0, b, t, :] += y[b, :]
            flag = flags_ref[tb * tc2 + t]

            @pl.when(flag == 1)
            def _():
                iota = jax.lax.broadcasted_iota(jnp.int32, (bsz, n), 1)
                xm = x_t
                sp = jnp.zeros((bsz, n), jnp.float32)
                for _ in range(TOPK):
                    m = jnp.max(xm, axis=1, keepdims=True)
                    cand = jnp.where(xm == m, iota, n)
                    first = jnp.min(cand, axis=1, keepdims=True)
                    hit = iota == first
                    sp = jnp.where(hit, xm, sp)
                    xm = jnp.where(hit, -jnp.inf, xm)
                hebb = jax.lax.dot_general(
                    sp, sp, (((0,), (0,)), ((), ())),
                    preferred_element_type=jnp.float32,
                    precision=jax.lax.Precision.HIGHEST)
                sig = sigma_ref[hh]
                hc = h_ref[hh]
                lam = LAMBDA_BASE * jnp.exp(-ALPHA * hc)
                sigma_ref[hh] = jnp.maximum(sig + ETA * hebb - lam * sig, 0.0)
                h_ref[hh] = hc + (hebb > 0).astype(jnp.float32)

            return carry

        jax.lax.fori_loop(0, tc2, step, 0)


def _k3_project(y_ref, w_ref, act_ref, o_ref):
    j = pl.program_id(1)
    _, _, tc3, n = y_ref.shape
    d = w_ref.shape[1]
    a = act_ref[j]

    @pl.when(a > 0)
    def _():
        y = y_ref[0, 0] + y_ref[1, 0]
        o = jnp.dot(y, w_ref[...], preferred_element_type=jnp.float32)
        o_ref[...] = o.reshape(1, 1, tc3, d)

    @pl.when(a == 0)
    def _():
        o_ref[...] = jnp.zeros_like(o_ref)


def kernel(Q, K, V, W_out):
    del K, V
    B, nh, T, N = Q.shape
    D = W_out.shape[0]
    f32 = jnp.float32

    TC1 = min(32, T)
    TC2 = min(256, T)
    TC3 = 512 if T % 512 == 0 else TC2
    n1 = T // TC1
    n2 = T // TC2
    n3 = T // TC3
    nhc = nh // 2

    nf = jnp.arange(N, dtype=f32)
    qq = jnp.floor(nf / 2.0) * 2.0
    freqs = 1.0 / (THETA ** (qq / N)) / (2.0 * jnp.pi)
    tf = jnp.arange(T, dtype=f32)
    ph = ((tf[:, None] * freqs[None, :]) % 1.0) * (2.0 * jnp.pi)
    cos_t = jnp.cos(ph)
    sin_t = jnp.sin(ph)

    qr, counts = pl.pallas_call(
        _k1_rope_count,
        grid=(n1,),
        in_specs=[
            pl.BlockSpec((B, nh, TC1, N), lambda tb: (0, 0, tb, 0)),
            pl.BlockSpec((TC1, N), lambda tb: (tb, 0)),
            pl.BlockSpec((TC1, N), lambda tb: (tb, 0)),
        ],
        out_specs=[
            pl.BlockSpec((nh, B, TC1, N), lambda tb: (0, 0, tb, 0)),
            pl.BlockSpec((1, 1, TC1), lambda tb: (tb, 0, 0)),
        ],
        out_shape=[
            jax.ShapeDtypeStruct((nh, B, T, N), f32),
            jax.ShapeDtypeStruct((n1, 1, TC1), jnp.int32),
        ],
        compiler_params=pltpu.CompilerParams(
            dimension_semantics=("parallel",),
            vmem_limit_bytes=56 * 1024 * 1024),
        name="rope_count",
    )(Q, cos_t, sin_t)

    total = f32(B * nh * N)
    do_t = ((counts.reshape(T).astype(f32) / total) <= ACT_THRESH)
    do_i = do_t.astype(jnp.int32)
    chunk_any = do_i.reshape(n2, TC2).max(axis=1)
    before = (jnp.cumsum(chunk_any) - chunk_any) > 0
    state = jnp.where(chunk_any == 1, 2,
                      jnp.where(before, 1, 0)).astype(jnp.int32)

    yagg = pl.pallas_call(
        _k2_scan,
        grid=(2, n2, nhc),
        in_specs=[
            pl.BlockSpec((1, B, TC2, N),
                         lambda c, tb, hh: (c * nhc + hh, 0, tb, 0)),
            pl.BlockSpec(memory_space=pltpu.SMEM),
            pl.BlockSpec(memory_space=pltpu.SMEM),
        ],
        out_specs=pl.BlockSpec((1, B, TC2, N),
                               lambda c, tb, hh: (c, 0, tb, 0)),
        out_shape=jax.ShapeDtypeStruct((2, B, T, N), f32),
        scratch_shapes=[
            pltpu.VMEM((nhc, N, N), f32),
            pltpu.VMEM((nhc, N, N), f32),
        ],
        compiler_params=pltpu.CompilerParams(
            dimension_semantics=("parallel", "arbitrary", "arbitrary")),
        name="hebb_scan",
    )(qr, do_i, state)

    act3 = state.reshape(n3, TC3 // TC2).max(axis=1)
    Wt = W_out.T
    out = pl.pallas_call(
        _k3_project,
        grid=(B, n3),
        in_specs=[
            pl.BlockSpec((2, 1, TC3, N), lambda b, j: (0, b, j, 0)),
            pl.BlockSpec((N, D), lambda b, j: (0, 0)),
            pl.BlockSpec(memory_space=pltpu.SMEM),
        ],
        out_specs=pl.BlockSpec((1, 1, TC3, D), lambda b, j: (b, 0, j, 0)),
        out_shape=jax.ShapeDtypeStruct((B, 1, T, D), f32),
        compiler_params=pltpu.CompilerParams(
            dimension_semantics=("parallel", "arbitrary")),
        name="headsum_project",
    )(yagg, Wt, act3)

    return out
